# baseline (speedup 1.0000x reference)
import functools

import jax
import jax.numpy as jnp
from jax import lax
from jax.experimental import pallas as pl
from jax.experimental.pallas import tpu as pltpu
from jax.experimental.pallas import tpu_sc as plsc

BATCH = 16384
DIM = 64
LANES = 16
NUM_CORES = 2
NUM_SUBCORES = 16
NW = NUM_CORES * NUM_SUBCORES
B_PER_W = BATCH // NW
CHUNK = 128
NCH = B_PER_W // CHUNK

_mesh = plsc.VectorSubcoreMesh(core_axis_name="c", subcore_axis_name="s")


@functools.partial(
    pl.kernel,
    mesh=_mesh,
    out_type=jax.ShapeDtypeStruct((BATCH, DIM), jnp.float32),
    scratch_types=[
        pltpu.VMEM((NCH, CHUNK), jnp.int32),
        pltpu.VMEM((NCH, CHUNK, DIM), jnp.float32),
        pltpu.VMEM((NCH, CHUNK, DIM), jnp.float32),
        pltpu.VMEM((NCH, CHUNK, DIM), jnp.float32),
        pltpu.SemaphoreType.DMA,
    ],
    compiler_params=pltpu.CompilerParams(use_tc_tiling_on_sc=False),
)
def _calibrate(x_hbm, u_hbm, scale_hbm, bias_hbm, out_hbm,
               idx_v, x_v, s_v, b_v, sem):
    wid = lax.axis_index("s") * NUM_CORES + lax.axis_index("c")
    base = wid * B_PER_W

    for j in range(NCH):
        pltpu.sync_copy(u_hbm.at[pl.ds(base + j * CHUNK, CHUNK)], idx_v.at[j])

    copies = []
    for j in range(NCH):
        copies.append(pltpu.async_copy(
            x_hbm.at[pl.ds(base + j * CHUNK, CHUNK)], x_v.at[j], sem))
        copies.append(pltpu.async_copy(
            scale_hbm.at[idx_v.at[j]], s_v.at[j], sem))
        copies.append(pltpu.async_copy(
            bias_hbm.at[idx_v.at[j]], b_v.at[j], sem))
    for c in copies:
        c.wait()

    def row_body(r, _):
        for j in range(NCH):
            for k in range(DIM // LANES):
                sl = pl.ds(k * LANES, LANES)
                x_v[j, r, sl] = x_v[j, r, sl] * s_v[j, r, sl] + b_v[j, r, sl]
        return 0

    lax.fori_loop(0, CHUNK, row_body, 0)

    for j in range(NCH):
        pltpu.sync_copy(x_v.at[j], out_hbm.at[pl.ds(base + j * CHUNK, CHUNK)])


def kernel(x, u, scale_weight, bias_weight):
    return _calibrate(x, u, scale_weight, bias_weight)


# speedup vs baseline: 1.5586x; 1.5586x over previous
import functools

import jax
import jax.numpy as jnp
from jax import lax
from jax.experimental import pallas as pl
from jax.experimental.pallas import tpu as pltpu
from jax.experimental.pallas import tpu_sc as plsc

BATCH = 16384
DIM = 64
LANES = 16
NUM_CORES = 2
NUM_SUBCORES = 16
NW = NUM_CORES * NUM_SUBCORES
B_PER_W = BATCH // NW
CHUNK = 128
NCH = B_PER_W // CHUNK

_mesh = plsc.VectorSubcoreMesh(core_axis_name="c", subcore_axis_name="s")


@functools.partial(
    pl.kernel,
    mesh=_mesh,
    out_type=jax.ShapeDtypeStruct((BATCH, DIM), jnp.float32),
    scratch_types=[
        pltpu.VMEM((NCH, CHUNK), jnp.int32),
        pltpu.VMEM((CHUNK, DIM), jnp.float32),
        pltpu.VMEM((CHUNK, DIM), jnp.float32),
        pltpu.VMEM((CHUNK, DIM), jnp.float32),
        pltpu.SemaphoreType.DMA,
    ],
    compiler_params=pltpu.CompilerParams(use_tc_tiling_on_sc=True),
)
def _calibrate(x_hbm, u_hbm, scale_hbm, bias_hbm, out_hbm,
               idx_v, x_v, s_v, b_v, sem):
    wid = lax.axis_index("s") * NUM_CORES + lax.axis_index("c")
    base = wid * B_PER_W

    for j in range(NCH):
        pltpu.sync_copy(u_hbm.at[pl.ds(base + j * CHUNK, CHUNK)], idx_v.at[j])

    for j in range(NCH):
        cx = pltpu.async_copy(
            x_hbm.at[pl.ds(base + j * CHUNK, CHUNK)], x_v, sem)

        def issue(g, _):
            v = idx_v[j, pl.ds(g * LANES, LANES)]
            for k in range(LANES):
                ui = v[k]
                r = g * LANES + k
                pltpu.make_async_copy(scale_hbm.at[ui], s_v.at[r], sem).start()
                pltpu.make_async_copy(bias_hbm.at[ui], b_v.at[r], sem).start()
            return 0

        lax.fori_loop(0, CHUNK // LANES, issue, 0)

        cx.wait()

        def drain(i, _):
            pltpu.make_async_copy(scale_hbm.at[0], s_v.at[0], sem).wait()
            pltpu.make_async_copy(bias_hbm.at[0], b_v.at[0], sem).wait()
            return 0

        lax.fori_loop(0, CHUNK, drain, 0)

        def row_body(r, _):
            for k in range(DIM // LANES):
                sl = pl.ds(k * LANES, LANES)
                x_v[r, sl] = x_v[r, sl] * s_v[r, sl] + b_v[r, sl]
            return 0

        lax.fori_loop(0, CHUNK, row_body, 0)

        pltpu.sync_copy(x_v, out_hbm.at[pl.ds(base + j * CHUNK, CHUNK)])


def kernel(x, u, scale_weight, bias_weight):
    return _calibrate(x, u, scale_weight, bias_weight)
